# baseline (speedup 1.0000x reference)
import functools

import jax
import jax.numpy as jnp
import numpy as np
from jax import lax
from jax.experimental import pallas as pl
from jax.experimental.pallas import tpu as pltpu
from jax.experimental.pallas import tpu_sc as plsc

_FIELD_DIMS = [100000] * 26
_NUM_FIELDS = len(_FIELD_DIMS)
_OFFSETS = np.concatenate(([0], np.cumsum(_FIELD_DIMS)[:-1])).astype(np.int32)

_B = 4096
_F = _NUM_FIELDS
_FPAD = 32
_D = 32
_NC, _NS = 2, 16
_NW = _NC * _NS
_BPW = _B // _NW
_CHUNK = 128


def _fm_body(idx_hbm, idxp_hbm, emb_hbm, lin_hbm, out_hbm,
             idx_v, idxp_v, rows_v, lin_v, tvals_v, out_v, esem, lsem):
    w = lax.axis_index("s") * _NC + lax.axis_index("c")

    pltpu.sync_copy(idx_hbm.at[w], idx_v)
    pltpu.sync_copy(idxp_hbm.at[w], idxp_v)

    emb_copies = []
    for j in range(_F):
        c = pltpu.async_copy(
            emb_hbm.at[idx_v.at[j]], rows_v.at[pl.ds(j * _CHUNK, _CHUNK)], esem)
        emb_copies.append(c)
    lin_copies = []
    for j in range(_FPAD):
        c = pltpu.async_copy(
            lin_hbm.at[idxp_v.at[j]], lin_v.at[pl.ds(j * _CHUNK, _CHUNK)], lsem)
        lin_copies.append(c)
    for c in emb_copies:
        c.wait()
    for c in lin_copies:
        c.wait()

    lane = lax.iota(jnp.int32, 16)
    padmask = lane < (_F - 16)
    zero = jnp.zeros((16,), jnp.float32)

    def row_body(r, carry):
        s0 = zero
        s1 = zero
        q0 = zero
        q1 = zero
        for f in range(_F):
            e0 = rows_v[r * _F + f, pl.ds(0, 16)]
            e1 = rows_v[r * _F + f, pl.ds(16, 16)]
            s0 = s0 + e0
            q0 = q0 + e0 * e0
            s1 = s1 + e1
            q1 = q1 + e1 * e1
        t = (s0 * s0 - q0) + (s1 * s1 - q1)
        l0 = plsc.load_gather(lin_v, [r * _FPAD + lane])
        l1 = plsc.load_gather(lin_v, [r * _FPAD + 16 + lane])
        l1 = jnp.where(padmask, l1, 0.0)
        tvals_v[r, :] = 0.5 * t + l0 + l1
        return carry

    lax.fori_loop(0, _BPW, row_body, 0)

    for g in range(_BPW // 16):
        rows16 = g * 16 + lane
        acc = zero
        for d in range(16):
            acc = acc + plsc.load_gather(
                tvals_v, [rows16, jnp.full((16,), d, jnp.int32)])
        out_v[pl.ds(g * 16, 16)] = acc

    pltpu.sync_copy(out_v, out_hbm.at[w])


_fm_kernel = functools.partial(
    pl.kernel,
    out_type=jax.ShapeDtypeStruct((_NW, _BPW), jnp.float32),
    mesh=plsc.VectorSubcoreMesh(core_axis_name="c", subcore_axis_name="s"),
    scratch_types=[
        pltpu.VMEM((_F, _CHUNK), jnp.int32),
        pltpu.VMEM((_FPAD, _CHUNK), jnp.int32),
        pltpu.VMEM((_BPW * _F, _D), jnp.float32),
        pltpu.VMEM((_BPW * _FPAD,), jnp.float32),
        pltpu.VMEM((_BPW, 16), jnp.float32),
        pltpu.VMEM((_BPW,), jnp.float32),
        pltpu.SemaphoreType.DMA,
        pltpu.SemaphoreType.DMA,
    ],
    compiler_params=pltpu.CompilerParams(
        needs_layout_passes=False, use_tc_tiling_on_sc=False),
)(_fm_body)


def kernel(x, emb_table, linear_table, bias):
    offsets = jnp.asarray(_OFFSETS)
    idx = x + offsets[None, :]
    idxp = jnp.concatenate([idx, idx[:, : _FPAD - _F]], axis=1)
    idx3 = idx.reshape(_NW, _F, _CHUNK)
    idxp3 = idxp.reshape(_NW, _FPAD, _CHUNK)
    lin_flat = linear_table.reshape(-1)
    out = _fm_kernel(idx3, idxp3, emb_table, lin_flat)
    return out.reshape(_B, 1) + bias
